# Initial kernel scaffold; baseline (speedup 1.0000x reference)
import jax
import jax.numpy as jnp
from jax.experimental import pallas as pl


def kernel(X, edge_index, W1, b1, W2, b2):
    raise NotImplementedError("write your pallas kernel here")


# speedup vs baseline: 6.2779x; 6.2779x over previous
import functools

import jax
import jax.numpy as jnp
from jax import lax
from jax.experimental import pallas as pl
from jax.experimental.pallas import tpu as pltpu
from jax.experimental.pallas import tpu_sc as plsc

N = 10000
E = 320000
D = 128

NC = 2
NS = 16
CHUNK = 128
ROWS_PER_TILE = 80
ROWS = NC * NS * ROWS_PER_TILE
E_PAD = ROWS * CHUNK
NPT = 624
ZCH = 104

_BN = 1000
_GRID = N // _BN


def _mesh():
  return plsc.VectorSubcoreMesh(
      core_axis_name="c", subcore_axis_name="s", num_cores=NC, num_subcores=NS
  )


DEG_RPT = ROWS // NS


def _deg_body(src2d, dst2d, out, idx_v, ones_v, stage_v, deg_sh, sem):
  cid = lax.axis_index("c")
  sid = lax.axis_index("s")

  @pl.loop(0, CHUNK)
  def _(r):
    for c in range(D // 16):
      ones_v[r, pl.ds(c * 16, 16)] = jnp.ones((16,), jnp.float32)

  @pl.loop(0, ZCH)
  def _(r):
    for c in range(D // 16):
      stage_v[r, pl.ds(c * 16, 16)] = jnp.zeros((16,), jnp.float32)

  for k in range(NPT // ZCH):
    pltpu.sync_copy(stage_v.at[pl.ds(0, ZCH)],
                    deg_sh.at[pl.ds(pl.multiple_of(sid * NPT + k * ZCH, 8), ZCH)])

  @pl.when(sid == NS - 1)
  def _():
    pltpu.sync_copy(stage_v.at[pl.ds(0, 16)], deg_sh.at[pl.ds(N - 16, 16)])

  plsc.subcore_barrier()

  for h in range(2):
    row_base = pl.multiple_of(sid * DEG_RPT + h * (DEG_RPT // 2), 8)

    @pl.when(cid == 0)
    def _():
      pltpu.sync_copy(src2d.at[pl.ds(row_base, DEG_RPT // 2)], idx_v)

    @pl.when(cid == 1)
    def _():
      pltpu.sync_copy(dst2d.at[pl.ds(row_base, DEG_RPT // 2)], idx_v)

    @pl.loop(0, DEG_RPT // 2)
    def _(j):
      idx = plsc.Indices(idx_v.at[j], ignored_value=-1)
      pltpu.sync_copy(ones_v, deg_sh.at[idx], add=True)

  plsc.subcore_barrier()

  def _copy_out(row0, nrows):
    row0 = pl.multiple_of(row0, 8)
    pltpu.sync_copy(deg_sh.at[pl.ds(row0, nrows)], stage_v.at[pl.ds(0, nrows)])
    pltpu.sync_copy(stage_v.at[pl.ds(0, nrows)],
                    out.at[pl.ds(pl.multiple_of(cid * N + row0, 8), nrows)])

  for k in range(NPT // ZCH):
    _copy_out(sid * NPT + k * ZCH, ZCH)

  @pl.when(sid == NS - 1)
  def _():
    _copy_out(N - 16, 16)


@jax.jit
def _deg_call(src2d, dst2d):
  return pl.kernel(
      _deg_body,
      out_type=jax.ShapeDtypeStruct((NC * N, D), jnp.float32),
      mesh=_mesh(),
      scratch_types=[
          pltpu.VMEM((DEG_RPT // 2, CHUNK), jnp.int32),
          pltpu.VMEM((CHUNK, D), jnp.float32),
          pltpu.VMEM((ZCH, D), jnp.float32),
          pltpu.VMEM_SHARED((N, D), jnp.float32),
          pltpu.SemaphoreType.DMA,
      ],
  )(src2d, dst2d)


def _agg_body(y, src2d, dst2d, out,
              src_v, dst_v, rows_v, acc_sh, sem):
  cid = lax.axis_index("c")
  sid = lax.axis_index("s")

  @pl.loop(0, CHUNK)
  def _(r):
    for c in range(D // 16):
      rows_v[r, pl.ds(c * 16, 16)] = jnp.zeros((16,), jnp.float32)
  for k in range(NPT // ZCH):
    pltpu.sync_copy(rows_v.at[pl.ds(0, ZCH)],
                    acc_sh.at[pl.ds(pl.multiple_of(sid * NPT + k * ZCH, 8), ZCH)])

  @pl.when(sid == NS - 1)
  def _():
    pltpu.sync_copy(rows_v.at[pl.ds(0, 16)], acc_sh.at[pl.ds(N - 16, 16)])

  row_base = pl.multiple_of((cid * NS + sid) * ROWS_PER_TILE, 8)
  pltpu.sync_copy(src2d.at[pl.ds(row_base, ROWS_PER_TILE)], src_v)
  pltpu.sync_copy(dst2d.at[pl.ds(row_base, ROWS_PER_TILE)], dst_v)
  plsc.subcore_barrier()

  @pl.loop(0, ROWS_PER_TILE)
  def _(j):
    idx_s = plsc.Indices(src_v.at[j], ignored_value=-1)
    idx_d = plsc.Indices(dst_v.at[j], ignored_value=-1)
    pltpu.async_copy(y.at[idx_s], rows_v, sem).wait()
    pltpu.sync_copy(rows_v, acc_sh.at[idx_d], add=True)

  plsc.subcore_barrier()

  def _copy_out(row0, nrows):
    row0 = pl.multiple_of(row0, 8)
    pltpu.sync_copy(acc_sh.at[pl.ds(row0, nrows)], rows_v.at[pl.ds(0, nrows)])
    pltpu.sync_copy(rows_v.at[pl.ds(0, nrows)],
                    out.at[pl.ds(pl.multiple_of(cid * N + row0, 8), nrows)])

  for k in range(NPT // ZCH):
    _copy_out(sid * NPT + k * ZCH, ZCH)

  @pl.when(sid == NS - 1)
  def _():
    _copy_out(N - 16, 16)


@jax.jit
def _agg_call(y, src2d, dst2d):
  return pl.kernel(
      _agg_body,
      out_type=jax.ShapeDtypeStruct((NC * N, D), jnp.float32),
      mesh=_mesh(),
      scratch_types=[
          pltpu.VMEM((ROWS_PER_TILE, CHUNK), jnp.int32),
          pltpu.VMEM((ROWS_PER_TILE, CHUNK), jnp.int32),
          pltpu.VMEM((CHUNK, D), jnp.float32),
          pltpu.VMEM_SHARED((N, D), jnp.float32),
          pltpu.SemaphoreType.DMA,
      ],
  )(y, src2d, dst2d)


def _rsqrt_deg(deg_ref):
  return lax.rsqrt(jnp.maximum(deg_ref[...][:, 0:1], 1.0))


def _p1_body(dego_ref, x_ref, w_ref, y_ref):
  ns = _rsqrt_deg(dego_ref)
  y_ref[...] = jnp.dot(x_ref[...] * ns, w_ref[...],
                       preferred_element_type=jnp.float32)


def _p2_body(lo_ref, hi_ref, degi_ref, dego_ref, b_ref, w_ref, y_ref):
  agg = lo_ref[...] + hi_ref[...]
  nd = _rsqrt_deg(degi_ref)
  z = jnp.maximum(agg * nd + b_ref[...], 0.0)
  ns = _rsqrt_deg(dego_ref)
  y_ref[...] = jnp.dot(z * ns, w_ref[...], preferred_element_type=jnp.float32)


def _p3_body(lo_ref, hi_ref, degi_ref, b_ref, out_ref):
  agg = lo_ref[...] + hi_ref[...]
  nd = _rsqrt_deg(degi_ref)
  out_ref[...] = agg * nd + b_ref[...]


def _row_spec(w):
  return pl.BlockSpec((_BN, w), lambda i: (i, 0))


def _full_spec(h, w):
  return pl.BlockSpec((h, w), lambda i: (0, 0))


def _p1_call(dego, x, w1):
  return pl.pallas_call(
      _p1_body,
      grid=(_GRID,),
      in_specs=[_row_spec(D), _row_spec(D), _full_spec(D, D)],
      out_specs=_row_spec(D),
      out_shape=jax.ShapeDtypeStruct((N, D), jnp.float32),
  )(dego, x, w1)


def _p2_call(lo, hi, degi, dego, b1, w2):
  return pl.pallas_call(
      _p2_body,
      grid=(_GRID,),
      in_specs=[_row_spec(D), _row_spec(D), _row_spec(D), _row_spec(D),
                _full_spec(1, D), _full_spec(D, D)],
      out_specs=_row_spec(D),
      out_shape=jax.ShapeDtypeStruct((N, D), jnp.float32),
  )(lo, hi, degi, dego, b1, w2)


def _p3_call(lo, hi, degi, b2):
  return pl.pallas_call(
      _p3_body,
      grid=(_GRID,),
      in_specs=[_row_spec(D), _row_spec(D), _row_spec(D), _full_spec(1, D)],
      out_specs=_row_spec(D),
      out_shape=jax.ShapeDtypeStruct((N, D), jnp.float32),
  )(lo, hi, degi, b2)


def kernel(X, edge_index, W1, b1, W2, b2):
  ep = jnp.pad(edge_index, ((0, 0), (0, E_PAD - E)), constant_values=-1)
  ep = ep.reshape(2, ROWS, CHUNK)
  src2d, dst2d = ep[0], ep[1]

  deg = _deg_call(src2d, dst2d)
  dego, degi = deg[:N], deg[N:]

  y1 = _p1_call(dego, X, W1)
  agg1 = _agg_call(y1, src2d, dst2d)
  y2 = _p2_call(agg1[:N], agg1[N:], degi, dego, b1.reshape(1, D), W2)
  agg2 = _agg_call(y2, src2d, dst2d)
  return _p3_call(agg2[:N], agg2[N:], degi, b2.reshape(1, D))


# speedup vs baseline: 7.5920x; 1.2093x over previous
import functools

import jax
import jax.numpy as jnp
from jax import lax
from jax.experimental import pallas as pl
from jax.experimental.pallas import tpu as pltpu
from jax.experimental.pallas import tpu_sc as plsc

N = 10000
E = 320000
D = 128

NC = 2
NS = 16
CHUNK = 128
ROWS_PER_TILE = 80
ROWS = NC * NS * ROWS_PER_TILE
E_PAD = ROWS * CHUNK
NPT = 624
ZCH = 104

_BN = 1000
_GRID = N // _BN


def _mesh():
  return plsc.VectorSubcoreMesh(
      core_axis_name="c", subcore_axis_name="s", num_cores=NC, num_subcores=NS
  )


DEG_RPT = ROWS // NS


def _deg_body(src2d, dst2d, out, idx_v, ones_v, stage_v, deg_sh, sem):
  cid = lax.axis_index("c")
  sid = lax.axis_index("s")

  @pl.loop(0, CHUNK)
  def _(r):
    for c in range(D // 16):
      ones_v[r, pl.ds(c * 16, 16)] = jnp.ones((16,), jnp.float32)

  @pl.loop(0, ZCH)
  def _(r):
    for c in range(D // 16):
      stage_v[r, pl.ds(c * 16, 16)] = jnp.zeros((16,), jnp.float32)

  for k in range(NPT // ZCH):
    pltpu.sync_copy(stage_v.at[pl.ds(0, ZCH)],
                    deg_sh.at[pl.ds(pl.multiple_of(sid * NPT + k * ZCH, 8), ZCH)])

  @pl.when(sid == NS - 1)
  def _():
    pltpu.sync_copy(stage_v.at[pl.ds(0, 16)], deg_sh.at[pl.ds(N - 16, 16)])

  plsc.subcore_barrier()

  for h in range(2):
    row_base = pl.multiple_of(sid * DEG_RPT + h * (DEG_RPT // 2), 8)

    @pl.when(cid == 0)
    def _():
      pltpu.sync_copy(src2d.at[pl.ds(row_base, DEG_RPT // 2)], idx_v)

    @pl.when(cid == 1)
    def _():
      pltpu.sync_copy(dst2d.at[pl.ds(row_base, DEG_RPT // 2)], idx_v)

    @pl.loop(0, DEG_RPT // 2)
    def _(j):
      idx = plsc.Indices(idx_v.at[j], ignored_value=-1)
      pltpu.sync_copy(ones_v, deg_sh.at[idx], add=True)

  plsc.subcore_barrier()

  def _copy_out(row0, nrows):
    row0 = pl.multiple_of(row0, 8)
    pltpu.sync_copy(deg_sh.at[pl.ds(row0, nrows)], stage_v.at[pl.ds(0, nrows)])
    pltpu.sync_copy(stage_v.at[pl.ds(0, nrows)],
                    out.at[pl.ds(pl.multiple_of(cid * N + row0, 8), nrows)])

  for k in range(NPT // ZCH):
    _copy_out(sid * NPT + k * ZCH, ZCH)

  @pl.when(sid == NS - 1)
  def _():
    _copy_out(N - 16, 16)


@jax.jit
def _deg_call(src2d, dst2d):
  return pl.kernel(
      _deg_body,
      out_type=jax.ShapeDtypeStruct((NC * N, D), jnp.float32),
      mesh=_mesh(),
      scratch_types=[
          pltpu.VMEM((DEG_RPT // 2, CHUNK), jnp.int32),
          pltpu.VMEM((CHUNK, D), jnp.float32),
          pltpu.VMEM((ZCH, D), jnp.float32),
          pltpu.VMEM_SHARED((N, D), jnp.float32),
          pltpu.SemaphoreType.DMA,
      ],
  )(src2d, dst2d)


GRP = 8
N_GROUPS = ROWS_PER_TILE // GRP


def _agg_body(y, src2d, dst2d, out,
              idx_v, rows_a, rows_b, acc_sh,
              gsem_a, gsem_b, ssem_a, ssem_b, isem):
  cid = lax.axis_index("c")
  sid = lax.axis_index("s")

  @pl.loop(0, CHUNK)
  def _(r):
    for c in range(D // 16):
      rows_a[r, pl.ds(c * 16, 16)] = jnp.zeros((16,), jnp.float32)
  for k in range(NPT // ZCH):
    pltpu.sync_copy(rows_a.at[pl.ds(0, ZCH)],
                    acc_sh.at[pl.ds(pl.multiple_of(sid * NPT + k * ZCH, 8), ZCH)])

  @pl.when(sid == NS - 1)
  def _():
    pltpu.sync_copy(rows_a.at[pl.ds(0, 16)], acc_sh.at[pl.ds(N - 16, 16)])

  row_base = pl.multiple_of((cid * NS + sid) * ROWS_PER_TILE, 8)

  def load_idx(g, slot):
    rb = pl.multiple_of(row_base + g * GRP, 8)
    pltpu.async_copy(src2d.at[pl.ds(rb, GRP)], idx_v.at[slot, 0], isem)
    pltpu.async_copy(dst2d.at[pl.ds(rb, GRP)], idx_v.at[slot, 1], isem)

  def drain_idx():
    for _ in range(2):
      pltpu.make_async_copy(src2d.at[pl.ds(0, GRP)], idx_v.at[0, 0],
                            isem).wait()

  def start_gather(slot, r, buf, gsem):
    idx_s = plsc.Indices(idx_v.at[slot, 0, r], ignored_value=-1)
    pltpu.async_copy(y.at[idx_s], buf, gsem)

  def wait_gather(buf, gsem):
    pltpu.make_async_copy(y.at[pl.ds(0, CHUNK)], buf, gsem).wait()

  def start_scatter(slot, r, buf, ssem):
    idx_d = plsc.Indices(idx_v.at[slot, 1, r], ignored_value=-1)
    pltpu.async_copy(buf, acc_sh.at[idx_d], ssem, add=True)

  def wait_scatter(buf, ssem):
    pltpu.make_async_copy(buf, acc_sh.at[pl.ds(0, CHUNK)], ssem).wait()

  plsc.subcore_barrier()

  load_idx(0, 0)
  drain_idx()
  start_gather(0, 0, rows_a, gsem_a)

  @pl.loop(0, N_GROUPS)
  def _(g):
    slot = lax.rem(g, 2)
    nslot = lax.rem(g + 1, 2)
    not_last = g != N_GROUPS - 1

    @pl.when(not_last)
    def _():
      load_idx(g + 1, nslot)

    for r in range(GRP):
      buf, gsem, ssem = ((rows_a, gsem_a, ssem_a) if r % 2 == 0
                         else (rows_b, gsem_b, ssem_b))
      obuf, ogsem = (rows_b, gsem_b) if r % 2 == 0 else (rows_a, gsem_a)
      wait_gather(buf, gsem)
      start_scatter(slot, r, buf, ssem)
      if r < GRP - 1:
        start_gather(slot, r + 1, obuf, ogsem)
      else:
        @pl.when(not_last)
        def _():
          drain_idx()
          start_gather(nslot, 0, obuf, ogsem)
      wait_scatter(buf, ssem)

  plsc.subcore_barrier()

  def _copy_out(row0, nrows):
    row0 = pl.multiple_of(row0, 8)
    pltpu.sync_copy(acc_sh.at[pl.ds(row0, nrows)], rows_a.at[pl.ds(0, nrows)])
    pltpu.sync_copy(rows_a.at[pl.ds(0, nrows)],
                    out.at[pl.ds(pl.multiple_of(cid * N + row0, 8), nrows)])

  for k in range(NPT // ZCH):
    _copy_out(sid * NPT + k * ZCH, ZCH)

  @pl.when(sid == NS - 1)
  def _():
    _copy_out(N - 16, 16)


@jax.jit
def _agg_call(y, src2d, dst2d):
  return pl.kernel(
      _agg_body,
      out_type=jax.ShapeDtypeStruct((NC * N, D), jnp.float32),
      mesh=_mesh(),
      scratch_types=[
          pltpu.VMEM((2, 2, GRP, CHUNK), jnp.int32),
          pltpu.VMEM((CHUNK, D), jnp.float32),
          pltpu.VMEM((CHUNK, D), jnp.float32),
          pltpu.VMEM_SHARED((N, D), jnp.float32),
          pltpu.SemaphoreType.DMA,
          pltpu.SemaphoreType.DMA,
          pltpu.SemaphoreType.DMA,
          pltpu.SemaphoreType.DMA,
          pltpu.SemaphoreType.DMA,
      ],
  )(y, src2d, dst2d)


def _rsqrt_deg(deg_ref):
  return lax.rsqrt(jnp.maximum(deg_ref[...][:, 0:1], 1.0))


def _p1_body(dego_ref, x_ref, w_ref, y_ref):
  ns = _rsqrt_deg(dego_ref)
  y_ref[...] = jnp.dot(x_ref[...] * ns, w_ref[...],
                       preferred_element_type=jnp.float32)


def _p2_body(lo_ref, hi_ref, degi_ref, dego_ref, b_ref, w_ref, y_ref):
  agg = lo_ref[...] + hi_ref[...]
  nd = _rsqrt_deg(degi_ref)
  z = jnp.maximum(agg * nd + b_ref[...], 0.0)
  ns = _rsqrt_deg(dego_ref)
  y_ref[...] = jnp.dot(z * ns, w_ref[...], preferred_element_type=jnp.float32)


def _p3_body(lo_ref, hi_ref, degi_ref, b_ref, out_ref):
  agg = lo_ref[...] + hi_ref[...]
  nd = _rsqrt_deg(degi_ref)
  out_ref[...] = agg * nd + b_ref[...]


def _row_spec(w):
  return pl.BlockSpec((_BN, w), lambda i: (i, 0))


def _full_spec(h, w):
  return pl.BlockSpec((h, w), lambda i: (0, 0))


def _p1_call(dego, x, w1):
  return pl.pallas_call(
      _p1_body,
      grid=(_GRID,),
      in_specs=[_row_spec(D), _row_spec(D), _full_spec(D, D)],
      out_specs=_row_spec(D),
      out_shape=jax.ShapeDtypeStruct((N, D), jnp.float32),
  )(dego, x, w1)


def _p2_call(lo, hi, degi, dego, b1, w2):
  return pl.pallas_call(
      _p2_body,
      grid=(_GRID,),
      in_specs=[_row_spec(D), _row_spec(D), _row_spec(D), _row_spec(D),
                _full_spec(1, D), _full_spec(D, D)],
      out_specs=_row_spec(D),
      out_shape=jax.ShapeDtypeStruct((N, D), jnp.float32),
  )(lo, hi, degi, dego, b1, w2)


def _p3_call(lo, hi, degi, b2):
  return pl.pallas_call(
      _p3_body,
      grid=(_GRID,),
      in_specs=[_row_spec(D), _row_spec(D), _row_spec(D), _full_spec(1, D)],
      out_specs=_row_spec(D),
      out_shape=jax.ShapeDtypeStruct((N, D), jnp.float32),
  )(lo, hi, degi, b2)


def kernel(X, edge_index, W1, b1, W2, b2):
  ep = jnp.pad(edge_index, ((0, 0), (0, E_PAD - E)), constant_values=-1)
  ep = ep.reshape(2, ROWS, CHUNK)
  src2d, dst2d = ep[0], ep[1]

  deg = _deg_call(src2d, dst2d)
  dego, degi = deg[:N], deg[N:]

  y1 = _p1_call(dego, X, W1)
  agg1 = _agg_call(y1, src2d, dst2d)
  y2 = _p2_call(agg1[:N], agg1[N:], degi, dego, b1.reshape(1, D), W2)
  agg2 = _agg_call(y2, src2d, dst2d)
  return _p3_call(agg2[:N], agg2[N:], degi, b2.reshape(1, D))


# speedup vs baseline: 7.5980x; 1.0008x over previous
import functools

import jax
import jax.numpy as jnp
from jax import lax
from jax.experimental import pallas as pl
from jax.experimental.pallas import tpu as pltpu
from jax.experimental.pallas import tpu_sc as plsc

N = 10000
E = 320000
D = 128

NC = 2
NS = 16
CHUNK = 128
ROWS_PER_TILE = 80
ROWS = NC * NS * ROWS_PER_TILE
E_PAD = ROWS * CHUNK
NPT = 624
ZCH = 104

_BN = 1000
_GRID = N // _BN


def _mesh():
  return plsc.VectorSubcoreMesh(
      core_axis_name="c", subcore_axis_name="s", num_cores=NC, num_subcores=NS
  )


DEG_RPT = ROWS // NS


def _deg_body(src2d, dst2d, out, idx_v, ones_v, stage_v, deg_sh, sem):
  cid = lax.axis_index("c")
  sid = lax.axis_index("s")

  @pl.loop(0, CHUNK)
  def _(r):
    for c in range(D // 16):
      ones_v[r, pl.ds(c * 16, 16)] = jnp.ones((16,), jnp.float32)

  @pl.loop(0, ZCH)
  def _(r):
    for c in range(D // 16):
      stage_v[r, pl.ds(c * 16, 16)] = jnp.zeros((16,), jnp.float32)

  for k in range(NPT // ZCH):
    pltpu.sync_copy(stage_v.at[pl.ds(0, ZCH)],
                    deg_sh.at[pl.ds(pl.multiple_of(sid * NPT + k * ZCH, 8), ZCH)])

  @pl.when(sid == NS - 1)
  def _():
    pltpu.sync_copy(stage_v.at[pl.ds(0, 16)], deg_sh.at[pl.ds(N - 16, 16)])

  plsc.subcore_barrier()

  for h in range(2):
    row_base = pl.multiple_of(sid * DEG_RPT + h * (DEG_RPT // 2), 8)

    @pl.when(cid == 0)
    def _():
      pltpu.sync_copy(src2d.at[pl.ds(row_base, DEG_RPT // 2)], idx_v)

    @pl.when(cid == 1)
    def _():
      pltpu.sync_copy(dst2d.at[pl.ds(row_base, DEG_RPT // 2)], idx_v)

    @pl.loop(0, DEG_RPT // 2 // 8)
    def _(q):
      for t in range(8):
        idx = plsc.Indices(idx_v.at[q * 8 + t], ignored_value=-1)
        pltpu.async_copy(ones_v, deg_sh.at[idx], sem, add=True)
      for t in range(8):
        pltpu.make_async_copy(ones_v, deg_sh.at[pl.ds(0, CHUNK)], sem).wait()

  plsc.subcore_barrier()

  def _copy_out(row0, nrows):
    row0 = pl.multiple_of(row0, 8)
    pltpu.sync_copy(deg_sh.at[pl.ds(row0, nrows)], stage_v.at[pl.ds(0, nrows)])
    pltpu.sync_copy(stage_v.at[pl.ds(0, nrows)],
                    out.at[pl.ds(pl.multiple_of(cid * N + row0, 8), nrows)])

  for k in range(NPT // ZCH):
    _copy_out(sid * NPT + k * ZCH, ZCH)

  @pl.when(sid == NS - 1)
  def _():
    _copy_out(N - 16, 16)


@jax.jit
def _deg_call(src2d, dst2d):
  return pl.kernel(
      _deg_body,
      out_type=jax.ShapeDtypeStruct((NC * N, D), jnp.float32),
      mesh=_mesh(),
      scratch_types=[
          pltpu.VMEM((DEG_RPT // 2, CHUNK), jnp.int32),
          pltpu.VMEM((CHUNK, D), jnp.float32),
          pltpu.VMEM((ZCH, D), jnp.float32),
          pltpu.VMEM_SHARED((N, D), jnp.float32),
          pltpu.SemaphoreType.DMA,
      ],
  )(src2d, dst2d)


GRP = 8
N_GROUPS = ROWS_PER_TILE // GRP


def _agg_body(y, src2d, dst2d, out,
              idx_v, rows_a, rows_b, acc_sh,
              gsem_a, gsem_b, ssem_a, ssem_b, isem):
  cid = lax.axis_index("c")
  sid = lax.axis_index("s")

  @pl.loop(0, CHUNK)
  def _(r):
    for c in range(D // 16):
      rows_a[r, pl.ds(c * 16, 16)] = jnp.zeros((16,), jnp.float32)
  for k in range(NPT // ZCH):
    pltpu.sync_copy(rows_a.at[pl.ds(0, ZCH)],
                    acc_sh.at[pl.ds(pl.multiple_of(sid * NPT + k * ZCH, 8), ZCH)])

  @pl.when(sid == NS - 1)
  def _():
    pltpu.sync_copy(rows_a.at[pl.ds(0, 16)], acc_sh.at[pl.ds(N - 16, 16)])

  row_base = pl.multiple_of((cid * NS + sid) * ROWS_PER_TILE, 8)

  def load_idx(g, slot):
    rb = pl.multiple_of(row_base + g * GRP, 8)
    pltpu.async_copy(src2d.at[pl.ds(rb, GRP)], idx_v.at[slot, 0], isem)
    pltpu.async_copy(dst2d.at[pl.ds(rb, GRP)], idx_v.at[slot, 1], isem)

  def drain_idx():
    for _ in range(2):
      pltpu.make_async_copy(src2d.at[pl.ds(0, GRP)], idx_v.at[0, 0],
                            isem).wait()

  def start_gather(slot, r, buf, gsem):
    idx_s = plsc.Indices(idx_v.at[slot, 0, r], ignored_value=-1)
    pltpu.async_copy(y.at[idx_s], buf, gsem)

  def wait_gather(buf, gsem):
    pltpu.make_async_copy(y.at[pl.ds(0, CHUNK)], buf, gsem).wait()

  def start_scatter(slot, r, buf, ssem):
    idx_d = plsc.Indices(idx_v.at[slot, 1, r], ignored_value=-1)
    pltpu.async_copy(buf, acc_sh.at[idx_d], ssem, add=True)

  def wait_scatter(buf, ssem):
    pltpu.make_async_copy(buf, acc_sh.at[pl.ds(0, CHUNK)], ssem).wait()

  plsc.subcore_barrier()

  load_idx(0, 0)
  drain_idx()
  start_gather(0, 0, rows_a, gsem_a)

  @pl.loop(0, N_GROUPS)
  def _(g):
    slot = lax.rem(g, 2)
    nslot = lax.rem(g + 1, 2)
    not_last = g != N_GROUPS - 1

    @pl.when(not_last)
    def _():
      load_idx(g + 1, nslot)

    for r in range(GRP):
      buf, gsem, ssem = ((rows_a, gsem_a, ssem_a) if r % 2 == 0
                         else (rows_b, gsem_b, ssem_b))
      obuf, ogsem = (rows_b, gsem_b) if r % 2 == 0 else (rows_a, gsem_a)
      wait_gather(buf, gsem)
      start_scatter(slot, r, buf, ssem)
      if r < GRP - 1:
        start_gather(slot, r + 1, obuf, ogsem)
      else:
        @pl.when(not_last)
        def _():
          drain_idx()
          start_gather(nslot, 0, obuf, ogsem)
      wait_scatter(buf, ssem)

  plsc.subcore_barrier()

  def _copy_out(row0, nrows):
    row0 = pl.multiple_of(row0, 8)
    pltpu.sync_copy(acc_sh.at[pl.ds(row0, nrows)], rows_a.at[pl.ds(0, nrows)])
    pltpu.sync_copy(rows_a.at[pl.ds(0, nrows)],
                    out.at[pl.ds(pl.multiple_of(cid * N + row0, 8), nrows)])

  for k in range(NPT // ZCH):
    _copy_out(sid * NPT + k * ZCH, ZCH)

  @pl.when(sid == NS - 1)
  def _():
    _copy_out(N - 16, 16)


@jax.jit
def _agg_call(y, src2d, dst2d):
  return pl.kernel(
      _agg_body,
      out_type=jax.ShapeDtypeStruct((NC * N, D), jnp.float32),
      mesh=_mesh(),
      scratch_types=[
          pltpu.VMEM((2, 2, GRP, CHUNK), jnp.int32),
          pltpu.VMEM((CHUNK, D), jnp.float32),
          pltpu.VMEM((CHUNK, D), jnp.float32),
          pltpu.VMEM_SHARED((N, D), jnp.float32),
          pltpu.SemaphoreType.DMA,
          pltpu.SemaphoreType.DMA,
          pltpu.SemaphoreType.DMA,
          pltpu.SemaphoreType.DMA,
          pltpu.SemaphoreType.DMA,
      ],
  )(y, src2d, dst2d)


def _rsqrt_deg(deg_ref):
  return lax.rsqrt(jnp.maximum(deg_ref[...][:, 0:1], 1.0))


def _p1_body(dego_ref, x_ref, w_ref, y_ref):
  ns = _rsqrt_deg(dego_ref)
  y_ref[...] = jnp.dot(x_ref[...] * ns, w_ref[...],
                       preferred_element_type=jnp.float32)


def _p2_body(lo_ref, hi_ref, degi_ref, dego_ref, b_ref, w_ref, y_ref):
  agg = lo_ref[...] + hi_ref[...]
  nd = _rsqrt_deg(degi_ref)
  z = jnp.maximum(agg * nd + b_ref[...], 0.0)
  ns = _rsqrt_deg(dego_ref)
  y_ref[...] = jnp.dot(z * ns, w_ref[...], preferred_element_type=jnp.float32)


def _p3_body(lo_ref, hi_ref, degi_ref, b_ref, out_ref):
  agg = lo_ref[...] + hi_ref[...]
  nd = _rsqrt_deg(degi_ref)
  out_ref[...] = agg * nd + b_ref[...]


def _row_spec(w):
  return pl.BlockSpec((_BN, w), lambda i: (i, 0))


def _full_spec(h, w):
  return pl.BlockSpec((h, w), lambda i: (0, 0))


def _p1_call(dego, x, w1):
  return pl.pallas_call(
      _p1_body,
      grid=(_GRID,),
      in_specs=[_row_spec(D), _row_spec(D), _full_spec(D, D)],
      out_specs=_row_spec(D),
      out_shape=jax.ShapeDtypeStruct((N, D), jnp.float32),
  )(dego, x, w1)


def _p2_call(lo, hi, degi, dego, b1, w2):
  return pl.pallas_call(
      _p2_body,
      grid=(_GRID,),
      in_specs=[_row_spec(D), _row_spec(D), _row_spec(D), _row_spec(D),
                _full_spec(1, D), _full_spec(D, D)],
      out_specs=_row_spec(D),
      out_shape=jax.ShapeDtypeStruct((N, D), jnp.float32),
  )(lo, hi, degi, dego, b1, w2)


def _p3_call(lo, hi, degi, b2):
  return pl.pallas_call(
      _p3_body,
      grid=(_GRID,),
      in_specs=[_row_spec(D), _row_spec(D), _row_spec(D), _full_spec(1, D)],
      out_specs=_row_spec(D),
      out_shape=jax.ShapeDtypeStruct((N, D), jnp.float32),
  )(lo, hi, degi, b2)


def kernel(X, edge_index, W1, b1, W2, b2):
  ep = jnp.pad(edge_index, ((0, 0), (0, E_PAD - E)), constant_values=-1)
  ep = ep.reshape(2, ROWS, CHUNK)
  src2d, dst2d = ep[0], ep[1]

  deg = _deg_call(src2d, dst2d)
  dego, degi = deg[:N], deg[N:]

  y1 = _p1_call(dego, X, W1)
  agg1 = _agg_call(y1, src2d, dst2d)
  y2 = _p2_call(agg1[:N], agg1[N:], degi, dego, b1.reshape(1, D), W2)
  agg2 = _agg_call(y2, src2d, dst2d)
  return _p3_call(agg2[:N], agg2[N:], degi, b2.reshape(1, D))


# speedup vs baseline: 10.0101x; 1.3175x over previous
import functools

import jax
import jax.numpy as jnp
from jax import lax
from jax.experimental import pallas as pl
from jax.experimental.pallas import tpu as pltpu
from jax.experimental.pallas import tpu_sc as plsc

N = 10000
E = 320000
D = 128

NC = 2
NS = 16
CHUNK = 128
ROWS_PER_TILE = 80
ROWS = NC * NS * ROWS_PER_TILE
E_PAD = ROWS * CHUNK
NPT = 624
ZCH = 104

_BN = 1000
_GRID = N // _BN


def _mesh():
  return plsc.VectorSubcoreMesh(
      core_axis_name="c", subcore_axis_name="s", num_cores=NC, num_subcores=NS
  )


DEG_RPT = ROWS // NS
HROWS = 80


def _deg_body(src2d, dst2d, iota80, out, idx_v, hist_v, iota_v, tbl_sh, sem):
  cid = lax.axis_index("c")
  sid = lax.axis_index("s")

  @pl.loop(0, HROWS)
  def _(r):
    for c in range(D // 16):
      hist_v[r, pl.ds(c * 16, 16)] = jnp.zeros((16,), jnp.float32)

  @pl.when(sid == 0)
  def _():
    pltpu.sync_copy(hist_v, tbl_sh)

  pltpu.sync_copy(iota80, iota_v)

  row_base = pl.multiple_of(sid * DEG_RPT, 8)

  @pl.when(cid == 0)
  def _():
    pltpu.sync_copy(src2d.at[pl.ds(row_base, DEG_RPT)], idx_v)

  @pl.when(cid == 1)
  def _():
    pltpu.sync_copy(dst2d.at[pl.ds(row_base, DEG_RPT)], idx_v)

  plsc.subcore_barrier()

  ones = jnp.ones((16,), jnp.float32)

  @pl.loop(0, DEG_RPT)
  def _(r):
    for c in range(CHUNK // 16):
      idx16 = idx_v[r, pl.ds(c * 16, 16)]
      mask = idx16 >= 0
      idx_c = jnp.maximum(idx16, 0)
      hi = lax.shift_right_logical(idx_c, 7)
      lo = lax.bitwise_and(idx_c, 127)
      plsc.addupdate_scatter(hist_v, [hi, lo], ones, mask=mask)

  idx_m = plsc.Indices(iota_v, ignored_value=-1)
  pltpu.sync_copy(hist_v, tbl_sh.at[idx_m], add=True)
  plsc.subcore_barrier()

  @pl.when(sid == 0)
  def _():
    pltpu.sync_copy(tbl_sh, hist_v)
    pltpu.sync_copy(hist_v,
                    out.at[pl.ds(pl.multiple_of(cid * HROWS, 8), HROWS)])


@jax.jit
def _deg_call(src2d, dst2d, iota80):
  return pl.kernel(
      _deg_body,
      out_type=jax.ShapeDtypeStruct((NC * HROWS, D), jnp.float32),
      mesh=_mesh(),
      compiler_params=pltpu.CompilerParams(needs_layout_passes=False),
      scratch_types=[
          pltpu.VMEM((DEG_RPT, CHUNK), jnp.int32),
          pltpu.VMEM((HROWS, D), jnp.float32),
          pltpu.VMEM((HROWS,), jnp.int32),
          pltpu.VMEM_SHARED((HROWS, D), jnp.float32),
          pltpu.SemaphoreType.DMA,
      ],
  )(src2d, dst2d, iota80)


GRP = 8
N_GROUPS = ROWS_PER_TILE // GRP


def _agg_body(y, src2d, dst2d, out,
              idx_v, rows_a, rows_b, acc_sh,
              gsem_a, gsem_b, ssem_a, ssem_b, isem):
  cid = lax.axis_index("c")
  sid = lax.axis_index("s")

  @pl.loop(0, CHUNK)
  def _(r):
    for c in range(D // 16):
      rows_a[r, pl.ds(c * 16, 16)] = jnp.zeros((16,), jnp.float32)
  for k in range(NPT // ZCH):
    pltpu.sync_copy(rows_a.at[pl.ds(0, ZCH)],
                    acc_sh.at[pl.ds(pl.multiple_of(sid * NPT + k * ZCH, 8), ZCH)])

  @pl.when(sid == NS - 1)
  def _():
    pltpu.sync_copy(rows_a.at[pl.ds(0, 16)], acc_sh.at[pl.ds(N - 16, 16)])

  row_base = pl.multiple_of((cid * NS + sid) * ROWS_PER_TILE, 8)

  def load_idx(g, slot):
    rb = pl.multiple_of(row_base + g * GRP, 8)
    pltpu.async_copy(src2d.at[pl.ds(rb, GRP)], idx_v.at[slot, 0], isem)
    pltpu.async_copy(dst2d.at[pl.ds(rb, GRP)], idx_v.at[slot, 1], isem)

  def drain_idx():
    for _ in range(2):
      pltpu.make_async_copy(src2d.at[pl.ds(0, GRP)], idx_v.at[0, 0],
                            isem).wait()

  def start_gather(slot, r, buf, gsem):
    idx_s = plsc.Indices(idx_v.at[slot, 0, r], ignored_value=-1)
    pltpu.async_copy(y.at[idx_s], buf, gsem)

  def wait_gather(buf, gsem):
    pltpu.make_async_copy(y.at[pl.ds(0, CHUNK)], buf, gsem).wait()

  def start_scatter(slot, r, buf, ssem):
    idx_d = plsc.Indices(idx_v.at[slot, 1, r], ignored_value=-1)
    pltpu.async_copy(buf, acc_sh.at[idx_d], ssem, add=True)

  def wait_scatter(buf, ssem):
    pltpu.make_async_copy(buf, acc_sh.at[pl.ds(0, CHUNK)], ssem).wait()

  plsc.subcore_barrier()

  load_idx(0, 0)
  drain_idx()
  start_gather(0, 0, rows_a, gsem_a)

  @pl.loop(0, N_GROUPS)
  def _(g):
    slot = lax.rem(g, 2)
    nslot = lax.rem(g + 1, 2)
    not_last = g != N_GROUPS - 1

    @pl.when(not_last)
    def _():
      load_idx(g + 1, nslot)

    for r in range(GRP):
      buf, gsem, ssem = ((rows_a, gsem_a, ssem_a) if r % 2 == 0
                         else (rows_b, gsem_b, ssem_b))
      obuf, ogsem = (rows_b, gsem_b) if r % 2 == 0 else (rows_a, gsem_a)
      wait_gather(buf, gsem)
      start_scatter(slot, r, buf, ssem)
      if r < GRP - 1:
        start_gather(slot, r + 1, obuf, ogsem)
      else:
        @pl.when(not_last)
        def _():
          drain_idx()
          start_gather(nslot, 0, obuf, ogsem)
      wait_scatter(buf, ssem)

  plsc.subcore_barrier()

  def _copy_out(row0, nrows):
    row0 = pl.multiple_of(row0, 8)
    pltpu.sync_copy(acc_sh.at[pl.ds(row0, nrows)], rows_a.at[pl.ds(0, nrows)])
    pltpu.sync_copy(rows_a.at[pl.ds(0, nrows)],
                    out.at[pl.ds(pl.multiple_of(cid * N + row0, 8), nrows)])

  for k in range(NPT // ZCH):
    _copy_out(sid * NPT + k * ZCH, ZCH)

  @pl.when(sid == NS - 1)
  def _():
    _copy_out(N - 16, 16)


@jax.jit
def _agg_call(y, src2d, dst2d):
  return pl.kernel(
      _agg_body,
      out_type=jax.ShapeDtypeStruct((NC * N, D), jnp.float32),
      mesh=_mesh(),
      scratch_types=[
          pltpu.VMEM((2, 2, GRP, CHUNK), jnp.int32),
          pltpu.VMEM((CHUNK, D), jnp.float32),
          pltpu.VMEM((CHUNK, D), jnp.float32),
          pltpu.VMEM_SHARED((N, D), jnp.float32),
          pltpu.SemaphoreType.DMA,
          pltpu.SemaphoreType.DMA,
          pltpu.SemaphoreType.DMA,
          pltpu.SemaphoreType.DMA,
          pltpu.SemaphoreType.DMA,
      ],
  )(y, src2d, dst2d)


def _rsqrt_deg(deg_ref):
  return lax.rsqrt(jnp.maximum(deg_ref[...], 1.0))


def _p1_body(dego_ref, x_ref, w_ref, y_ref):
  ns = _rsqrt_deg(dego_ref)
  y_ref[...] = jnp.dot(x_ref[...] * ns, w_ref[...],
                       preferred_element_type=jnp.float32)


def _p2_body(lo_ref, hi_ref, degi_ref, dego_ref, b_ref, w_ref, y_ref):
  agg = lo_ref[...] + hi_ref[...]
  nd = _rsqrt_deg(degi_ref)
  z = jnp.maximum(agg * nd + b_ref[...], 0.0)
  ns = _rsqrt_deg(dego_ref)
  y_ref[...] = jnp.dot(z * ns, w_ref[...], preferred_element_type=jnp.float32)


def _p3_body(lo_ref, hi_ref, degi_ref, b_ref, out_ref):
  agg = lo_ref[...] + hi_ref[...]
  nd = _rsqrt_deg(degi_ref)
  out_ref[...] = agg * nd + b_ref[...]


def _row_spec(w):
  return pl.BlockSpec((_BN, w), lambda i: (i, 0))


def _full_spec(h, w):
  return pl.BlockSpec((h, w), lambda i: (0, 0))


def _p1_call(dego, x, w1):
  return pl.pallas_call(
      _p1_body,
      grid=(_GRID,),
      in_specs=[_row_spec(1), _row_spec(D), _full_spec(D, D)],
      out_specs=_row_spec(D),
      out_shape=jax.ShapeDtypeStruct((N, D), jnp.float32),
  )(dego, x, w1)


def _p2_call(lo, hi, degi, dego, b1, w2):
  return pl.pallas_call(
      _p2_body,
      grid=(_GRID,),
      in_specs=[_row_spec(D), _row_spec(D), _row_spec(1), _row_spec(1),
                _full_spec(1, D), _full_spec(D, D)],
      out_specs=_row_spec(D),
      out_shape=jax.ShapeDtypeStruct((N, D), jnp.float32),
  )(lo, hi, degi, dego, b1, w2)


def _p3_call(lo, hi, degi, b2):
  return pl.pallas_call(
      _p3_body,
      grid=(_GRID,),
      in_specs=[_row_spec(D), _row_spec(D), _row_spec(1), _full_spec(1, D)],
      out_specs=_row_spec(D),
      out_shape=jax.ShapeDtypeStruct((N, D), jnp.float32),
  )(lo, hi, degi, b2)


def kernel(X, edge_index, W1, b1, W2, b2):
  ep = jnp.pad(edge_index, ((0, 0), (0, E_PAD - E)), constant_values=-1)
  ep = ep.reshape(2, ROWS, CHUNK)
  src2d, dst2d = ep[0], ep[1]

  iota80 = jnp.arange(HROWS, dtype=jnp.int32)
  deg = _deg_call(src2d, dst2d, iota80)
  degf = deg.reshape(NC, HROWS * D)
  dego = degf[0, :N].reshape(N, 1)
  degi = degf[1, :N].reshape(N, 1)

  y1 = _p1_call(dego, X, W1)
  agg1 = _agg_call(y1, src2d, dst2d)
  y2 = _p2_call(agg1[:N], agg1[N:], degi, dego, b1.reshape(1, D), W2)
  agg2 = _agg_call(y2, src2d, dst2d)
  return _p3_call(agg2[:N], agg2[N:], degi, b2.reshape(1, D))
